# Initial kernel scaffold; baseline (speedup 1.0000x reference)
import jax
import jax.numpy as jnp
from jax.experimental import pallas as pl


def kernel(x, edge_index, Wh, bh, Wo, bo):
    raise NotImplementedError("write your pallas kernel here")


# speedup vs baseline: 3.8038x; 3.8038x over previous
import functools

import jax
import jax.numpy as jnp
from jax import lax
from jax.experimental import pallas as pl
from jax.experimental.pallas import tpu as pltpu
from jax.experimental.pallas import tpu_sc as plsc

N_NODES = 10000
N_EDGES = 320000
D_FEAT = 128
N_UNITS = 128
OUT_FEATS = 40
N_LAYERS = 6

NC = 2
NS = 16
NW = NC * NS
CH = 128
NCH = 79
EPT = CH * NCH
E_PAD = EPT * NW
R = 10240
RPS = R // NS
DUMMY = R - 1
BM = 1024


_sc_mesh = plsc.VectorSubcoreMesh(core_axis_name="c", subcore_axis_name="s")


@functools.partial(
    pl.kernel,
    mesh=_sc_mesh,
    out_type=jax.ShapeDtypeStruct((NC, R, N_UNITS), jnp.float32),
    scratch_types=[
        pltpu.VMEM((CH, N_UNITS), jnp.float32),
        pltpu.VMEM((CH,), jnp.int32),
        pltpu.VMEM((CH,), jnp.int32),
        pltpu.VMEM_SHARED((R, N_UNITS), jnp.float32),
        pltpu.SemaphoreType.DMA,
    ],
)
def _agg_sc(s_hbm, src_hbm, dst_hbm, out_hbm, rows_v, isrc_v, idst_v, acc_sh, sem):
    cid = lax.axis_index("c")
    sid = lax.axis_index("s")
    wid = cid * NS + sid

    z16 = jnp.zeros((16,), jnp.float32)

    def _zr(i, carry):
        r = i // 8
        c = (i % 8) * 16
        rows_v[r, pl.ds(c, 16)] = z16
        return carry

    lax.fori_loop(0, CH * 8, _zr, 0)
    base_rows = sid * RPS
    for k in range(RPS // CH):
        pltpu.sync_copy(rows_v, acc_sh.at[pl.ds(base_rows + k * CH, CH)])
    plsc.subcore_barrier()

    ebase = wid * EPT

    def _chunk(j, carry):
        off = ebase + j * CH
        pltpu.sync_copy(src_hbm.at[pl.ds(off, CH)], isrc_v)
        pltpu.sync_copy(dst_hbm.at[pl.ds(off, CH)], idst_v)
        pltpu.async_copy(s_hbm.at[isrc_v], rows_v, sem).wait()
        pltpu.sync_copy(rows_v, acc_sh.at[idst_v], add=True)
        return carry

    lax.fori_loop(0, NCH, _chunk, 0)
    plsc.subcore_barrier()

    for k in range(RPS // CH):
        off = base_rows + k * CH
        pltpu.sync_copy(acc_sh.at[pl.ds(off, CH)], rows_v)
        pltpu.sync_copy(rows_v, out_hbm.at[cid, pl.ds(off, CH)])


@functools.partial(
    pl.kernel,
    mesh=_sc_mesh,
    out_type=jax.ShapeDtypeStruct((NC, R, N_UNITS), jnp.float32),
    scratch_types=[
        pltpu.VMEM((CH, N_UNITS), jnp.float32),
        pltpu.VMEM((CH, N_UNITS), jnp.float32),
        pltpu.VMEM((CH,), jnp.int32),
        pltpu.VMEM((CH,), jnp.int32),
        pltpu.VMEM_SHARED((R, N_UNITS), jnp.float32),
    ],
)
def _deg_sc(src_hbm, dst_hbm, out_hbm, bufa_v, bufb_v, isrc_v, idst_v, h_sh):
    cid = lax.axis_index("c")
    sid = lax.axis_index("s")
    wid = cid * NS + sid

    z16 = jnp.zeros((16,), jnp.float32)
    o16 = jnp.ones((16,), jnp.float32)

    def _fill(i, carry):
        r = i // 8
        cc = i % 8
        col = cc * 16
        lo = (cc < 4).astype(jnp.float32)
        bufa_v[r, pl.ds(col, 16)] = z16
        bufb_v[r, pl.ds(col, 16)] = o16 * (1.0 - lo)
        return carry

    lax.fori_loop(0, CH * 8, _fill, 0)
    base_rows = sid * RPS
    for k in range(RPS // CH):
        pltpu.sync_copy(bufa_v, h_sh.at[pl.ds(base_rows + k * CH, CH)])

    def _fill2(i, carry):
        r = i // 8
        cc = i % 8
        col = cc * 16
        bufa_v[r, pl.ds(col, 16)] = o16 * (cc < 4).astype(jnp.float32)
        return carry

    lax.fori_loop(0, CH * 8, _fill2, 0)
    plsc.subcore_barrier()

    ebase = wid * EPT

    def _chunk(j, carry):
        off = ebase + j * CH
        pltpu.sync_copy(src_hbm.at[pl.ds(off, CH)], isrc_v)
        pltpu.sync_copy(dst_hbm.at[pl.ds(off, CH)], idst_v)
        pltpu.sync_copy(bufa_v, h_sh.at[isrc_v], add=True)
        pltpu.sync_copy(bufb_v, h_sh.at[idst_v], add=True)
        return carry

    lax.fori_loop(0, NCH, _chunk, 0)
    plsc.subcore_barrier()

    for k in range(RPS // CH):
        off = base_rows + k * CH
        pltpu.sync_copy(h_sh.at[pl.ds(off, CH)], bufb_v)
        pltpu.sync_copy(bufb_v, out_hbm.at[cid, pl.ds(off, CH)])


def _norm_body(hist_ref, x_ref, ns_ref, nd_ref, s0_ref):
    hb = hist_ref[...]
    degs = hb[0, :, 0:1] + hb[1, :, 0:1]
    degd = hb[0, :, 64:65] + hb[1, :, 64:65]
    i = pl.program_id(0)
    rid = lax.broadcasted_iota(jnp.int32, (BM, 1), 0) + i * BM
    mask = rid < N_NODES
    ns = jnp.where(mask, lax.rsqrt(jnp.clip(degs, 1.0, None)), 0.0)
    nd = jnp.where(mask, lax.rsqrt(jnp.clip(degd, 1.0, None)), 0.0)
    ns_ref[...] = ns
    nd_ref[...] = nd
    s0_ref[...] = x_ref[...] * ns


_norm_tc = pl.pallas_call(
    _norm_body,
    grid=(R // BM,),
    in_specs=[
        pl.BlockSpec((NC, BM, N_UNITS), lambda i: (0, i, 0)),
        pl.BlockSpec((BM, D_FEAT), lambda i: (i, 0)),
    ],
    out_specs=[
        pl.BlockSpec((BM, 1), lambda i: (i, 0)),
        pl.BlockSpec((BM, 1), lambda i: (i, 0)),
        pl.BlockSpec((BM, D_FEAT), lambda i: (i, 0)),
    ],
    out_shape=[
        jax.ShapeDtypeStruct((R, 1), jnp.float32),
        jax.ShapeDtypeStruct((R, 1), jnp.float32),
        jax.ShapeDtypeStruct((R, D_FEAT), jnp.float32),
    ],
)


def _layer_body(last, m2_ref, nd_ref, ns_ref, w_ref, b_ref, mprev_ref,
                s_ref, mnew_ref):
    m = m2_ref[0] + m2_ref[1]
    g = m * nd_ref[...]
    h = jnp.dot(g, w_ref[...], preferred_element_type=jnp.float32) + b_ref[...]
    h = jnp.maximum(h, 0.0)
    mn = jnp.maximum(mprev_ref[...], h)
    mnew_ref[...] = mn
    s_ref[...] = (mn if last else h) * ns_ref[...]


def _make_layer_tc(last):
    return pl.pallas_call(
        functools.partial(_layer_body, last),
        grid=(R // BM,),
        in_specs=[
            pl.BlockSpec((NC, BM, N_UNITS), lambda i: (0, i, 0)),
            pl.BlockSpec((BM, 1), lambda i: (i, 0)),
            pl.BlockSpec((BM, 1), lambda i: (i, 0)),
            pl.BlockSpec((N_UNITS, N_UNITS), lambda i: (0, 0)),
            pl.BlockSpec((1, N_UNITS), lambda i: (0, 0)),
            pl.BlockSpec((BM, N_UNITS), lambda i: (i, 0)),
        ],
        out_specs=[
            pl.BlockSpec((BM, N_UNITS), lambda i: (i, 0)),
            pl.BlockSpec((BM, N_UNITS), lambda i: (i, 0)),
        ],
        out_shape=[
            jax.ShapeDtypeStruct((R, N_UNITS), jnp.float32),
            jax.ShapeDtypeStruct((R, N_UNITS), jnp.float32),
        ],
    )


_layer_tc = _make_layer_tc(False)
_layer_tc_last = _make_layer_tc(True)


def _final_body(m2_ref, nd_ref, wo_ref, bo_ref, o_ref):
    m = m2_ref[0] + m2_ref[1]
    g = m * nd_ref[...]
    z = jnp.dot(g, wo_ref[...], preferred_element_type=jnp.float32) + bo_ref[...]
    zm = jnp.max(z, axis=1, keepdims=True)
    lse = jnp.log(jnp.sum(jnp.exp(z - zm), axis=1, keepdims=True)) + zm
    o_ref[...] = z - lse


_final_tc = pl.pallas_call(
    _final_body,
    grid=(R // BM,),
    in_specs=[
        pl.BlockSpec((NC, BM, N_UNITS), lambda i: (0, i, 0)),
        pl.BlockSpec((BM, 1), lambda i: (i, 0)),
        pl.BlockSpec((N_UNITS, OUT_FEATS), lambda i: (0, 0)),
        pl.BlockSpec((1, OUT_FEATS), lambda i: (0, 0)),
    ],
    out_specs=pl.BlockSpec((BM, OUT_FEATS), lambda i: (i, 0)),
    out_shape=jax.ShapeDtypeStruct((R, OUT_FEATS), jnp.float32),
)


def kernel(x, edge_index, Wh, bh, Wo, bo):
    pad = E_PAD - N_EDGES
    fill = jnp.full((pad,), DUMMY, jnp.int32)
    srcp = jnp.concatenate([edge_index[0], fill])
    dstp = jnp.concatenate([edge_index[1], fill])

    hist = _deg_sc(srcp, dstp)
    x_pad = jnp.pad(x, ((0, R - N_NODES), (0, 0)))
    ns, nd, s = _norm_tc(hist, x_pad)

    m_run = jnp.full((R, N_UNITS), -jnp.inf, jnp.float32)
    for i in range(N_LAYERS):
        m2 = _agg_sc(s, srcp, dstp)
        layer = _layer_tc_last if i == N_LAYERS - 1 else _layer_tc
        s, m_run = layer(m2, nd, ns, Wh[i], bh[i][None, :], m_run)

    m2 = _agg_sc(s, srcp, dstp)
    out = _final_tc(m2, nd, Wo, bo[None, :])
    return out[:N_NODES]
